# Initial kernel scaffold; baseline (speedup 1.0000x reference)
import jax
import jax.numpy as jnp
from jax.experimental import pallas as pl


def kernel(x, edge_indices, W0, b0, W1, b1, W2, b2):
    raise NotImplementedError("write your pallas kernel here")


# speedup vs baseline: 20.0211x; 20.0211x over previous
import functools

import jax
import jax.numpy as jnp
from jax import lax
from jax.experimental import pallas as pl
from jax.experimental.pallas import tpu as pltpu
from jax.experimental.pallas import tpu_sc as plsc

_N = 10000
_D = 128
_E = 320000
_L = 3

_NC = 2
_NS = 16
_NW = _NC * _NS

_LPAD = 10240
_BINS = _L * _LPAD
_EPW = _E // _NW
_K = 128
_NFULL = _EPW // _K
_TAIL = _EPW - _NFULL * _K
_APAD = 10240

_mesh = plsc.VectorSubcoreMesh(core_axis_name="c", subcore_axis_name="s")
_sc_params = pltpu.CompilerParams(needs_layout_passes=False)


@functools.partial(
    pl.kernel,
    mesh=_mesh,
    out_type=jax.ShapeDtypeStruct((_NW * _BINS,), jnp.float32),
    compiler_params=_sc_params,
    scratch_types=[
        pltpu.VMEM((_BINS,), jnp.float32),
        pltpu.VMEM((2000,), jnp.int32),
    ],
)
def _deg_kernel(ei_hbm, out_hbm, bins, estage):
    c = lax.axis_index("c")
    s = lax.axis_index("s")
    wid = c * _NS + s

    zeros16 = jnp.zeros((16,), jnp.float32)
    ones16 = jnp.ones((16,), jnp.float32)

    def zb(i, carry):
        bins[pl.ds(i * 16, 16)] = zeros16
        return carry

    lax.fori_loop(0, _BINS // 16, zb, 0)

    for l in range(_L):
        for st in range(5):
            pltpu.sync_copy(
                ei_hbm.at[pl.ds(l * _E + wid * _EPW + st * 2000, 2000)],
                estage,
            )

            def hb(i, carry):
                idx = estage[pl.ds(i * 16, 16)] + l * _LPAD
                plsc.addupdate_scatter(bins, [idx], ones16)
                return carry

            lax.fori_loop(0, 125, hb, 0)

    pltpu.sync_copy(bins, out_hbm.at[pl.ds(wid * _BINS, _BINS)])


@functools.partial(
    pl.kernel,
    mesh=_mesh,
    out_type=jax.ShapeDtypeStruct((_NC * _N, _D), jnp.float32),
    compiler_params=_sc_params,
    scratch_types=[
        pltpu.VMEM((_K,), jnp.int32),
        pltpu.VMEM((_K,), jnp.int32),
        pltpu.VMEM((_TAIL,), jnp.int32),
        pltpu.VMEM((_TAIL,), jnp.int32),
        pltpu.VMEM((_K, _D), jnp.float32),
        pltpu.VMEM((16, _D), jnp.float32),
        pltpu.VMEM_SHARED((_APAD, _D), jnp.float32),
        pltpu.SemaphoreType.DMA,
    ],
)
def _scatter_kernel(g_hbm, src_hbm, dst_hbm, out_hbm,
                    colv, rowv, colt, rowt, rows, zbuf, acc, sem):
    c = lax.axis_index("c")
    s = lax.axis_index("s")
    wid = c * _NS + s

    zeros16 = jnp.zeros((16,), jnp.float32)
    for r in range(16):
        for j in range(_D // 16):
            zbuf[r, pl.ds(j * 16, 16)] = zeros16

    stripe = _APAD // _NS

    def zs(i, carry):
        pltpu.sync_copy(zbuf, acc.at[pl.ds(s * stripe + i * 16, 16)])
        return carry

    lax.fori_loop(0, stripe // 16, zs, 0)
    plsc.subcore_barrier()

    base = wid * _EPW

    def body(i, carry):
        off = base + i * _K
        pltpu.sync_copy(src_hbm.at[pl.ds(off, _K)], colv)
        pltpu.async_copy(g_hbm.at[colv], rows, sem).wait()
        pltpu.sync_copy(dst_hbm.at[pl.ds(off, _K)], rowv)
        pltpu.sync_copy(rows, acc.at[rowv], add=True)
        return carry

    lax.fori_loop(0, _NFULL, body, 0)

    offt = base + _NFULL * _K
    pltpu.sync_copy(src_hbm.at[pl.ds(offt, _TAIL)], colt)
    pltpu.async_copy(g_hbm.at[colt], rows.at[pl.ds(0, _TAIL)], sem).wait()
    pltpu.sync_copy(dst_hbm.at[pl.ds(offt, _TAIL)], rowt)
    pltpu.sync_copy(rows.at[pl.ds(0, _TAIL)], acc.at[rowt], add=True)

    plsc.subcore_barrier()

    pltpu.sync_copy(
        acc.at[pl.ds(s * 624, 624)],
        out_hbm.at[pl.ds(c * _N + s * 624, 624)],
    )

    @pl.when(s == _NS - 1)
    def _wb_tail():
        pltpu.sync_copy(
            acc.at[pl.ds(624 * _NS, _N - 624 * _NS)],
            out_hbm.at[pl.ds(c * _N + 624 * _NS, _N - 624 * _NS)],
        )


_BN = 1000
_GRID = _N // _BN


def _dinv_body(degp_ref, o_ref):
    deg = jnp.sum(degp_ref[...], axis=0, keepdims=True) + 1.0
    o_ref[...] = lax.rsqrt(deg)


def _dinv_tc(degp):
    return pl.pallas_call(
        _dinv_body,
        grid=(8,),
        in_specs=[pl.BlockSpec((_NW, _BINS // 8), lambda i: (0, i))],
        out_specs=pl.BlockSpec((1, _BINS // 8), lambda i: (0, i)),
        out_shape=jax.ShapeDtypeStruct((1, _BINS), jnp.float32),
    )(degp)


def _t0_body(x_ref, w_ref, dv_ref, o_ref):
    h = jnp.dot(x_ref[...], w_ref[...], preferred_element_type=jnp.float32)
    o_ref[...] = dv_ref[...] * h


def _t0(x, w, dvc):
    return pl.pallas_call(
        _t0_body,
        grid=(_GRID,),
        in_specs=[
            pl.BlockSpec((_BN, _D), lambda i: (i, 0)),
            pl.BlockSpec((_D, _D), lambda i: (0, 0)),
            pl.BlockSpec((_BN, 1), lambda i: (i, 0)),
        ],
        out_specs=pl.BlockSpec((_BN, _D), lambda i: (i, 0)),
        out_shape=jax.ShapeDtypeStruct((_N, _D), jnp.float32),
    )(x, w, dvc)


def _tmid_body(pa_ref, pb_ref, g_ref, dva_ref, b_ref, w_ref, dvb_ref, o_ref):
    t = dva_ref[...] * (pa_ref[...] + pb_ref[...] + g_ref[...]) + b_ref[...]
    x1 = jnp.maximum(t, 0.0)
    h = jnp.dot(x1, w_ref[...], preferred_element_type=jnp.float32)
    o_ref[...] = dvb_ref[...] * h


def _tmid(pa, pb, g, dvac, b, w, dvbc):
    return pl.pallas_call(
        _tmid_body,
        grid=(_GRID,),
        in_specs=[
            pl.BlockSpec((_BN, _D), lambda i: (i, 0)),
            pl.BlockSpec((_BN, _D), lambda i: (i, 0)),
            pl.BlockSpec((_BN, _D), lambda i: (i, 0)),
            pl.BlockSpec((_BN, 1), lambda i: (i, 0)),
            pl.BlockSpec((1, _D), lambda i: (0, 0)),
            pl.BlockSpec((_D, _D), lambda i: (0, 0)),
            pl.BlockSpec((_BN, 1), lambda i: (i, 0)),
        ],
        out_specs=pl.BlockSpec((_BN, _D), lambda i: (i, 0)),
        out_shape=jax.ShapeDtypeStruct((_N, _D), jnp.float32),
    )(pa, pb, g, dvac, b, w, dvbc)


def _t3_body(pa_ref, pb_ref, g_ref, dva_ref, b_ref, o_ref):
    t = dva_ref[...] * (pa_ref[...] + pb_ref[...] + g_ref[...]) + b_ref[...]
    o_ref[...] = jnp.maximum(t, 0.0)


def _t3(pa, pb, g, dvac, b):
    return pl.pallas_call(
        _t3_body,
        grid=(_GRID,),
        in_specs=[
            pl.BlockSpec((_BN, _D), lambda i: (i, 0)),
            pl.BlockSpec((_BN, _D), lambda i: (i, 0)),
            pl.BlockSpec((_BN, _D), lambda i: (i, 0)),
            pl.BlockSpec((_BN, 1), lambda i: (i, 0)),
            pl.BlockSpec((1, _D), lambda i: (0, 0)),
        ],
        out_specs=pl.BlockSpec((_BN, _D), lambda i: (i, 0)),
        out_shape=jax.ShapeDtypeStruct((_N, _D), jnp.float32),
    )(pa, pb, g, dvac, b)


def kernel(x, edge_indices, W0, b0, W1, b1, W2, b2):
    Ws = [W0, W1, W2]
    bs = [b0.reshape(1, _D), b1.reshape(1, _D), b2.reshape(1, _D)]

    ei1cat = edge_indices[:, 1, :].reshape(_L * _E)
    degp = _deg_kernel(ei1cat)
    dinv = _dinv_tc(degp.reshape(_NW, _BINS)).reshape(_L, _LPAD)
    dcols = [dinv[l, :_N].reshape(_N, 1) for l in range(_L)]

    g = _t0(x, Ws[0], dcols[0])
    out = None
    for l in range(_L):
        p = _scatter_kernel(g, edge_indices[l, 0], edge_indices[l, 1])
        pa, pb = p[:_N], p[_N:]
        if l + 1 < _L:
            g = _tmid(pa, pb, g, dcols[l], bs[l], Ws[l + 1], dcols[l + 1])
        else:
            out = _t3(pa, pb, g, dcols[l], bs[l])
    return out


# speedup vs baseline: 28.6581x; 1.4314x over previous
import functools

import jax
import jax.numpy as jnp
from jax import lax
from jax.experimental import pallas as pl
from jax.experimental.pallas import tpu as pltpu
from jax.experimental.pallas import tpu_sc as plsc

_N = 10000
_D = 128
_E = 320000
_L = 3

_NC = 2
_NS = 16
_NW = _NC * _NS

_LPAD = 10240
_BINS = _L * _LPAD
_EPW = _E // _NW
_K = 128
_NFULL = _EPW // _K
_TAIL = _EPW - _NFULL * _K
_APAD = 10240

_mesh = plsc.VectorSubcoreMesh(core_axis_name="c", subcore_axis_name="s")
_sc_params = pltpu.CompilerParams(needs_layout_passes=False)


@functools.partial(
    pl.kernel,
    mesh=_mesh,
    out_type=jax.ShapeDtypeStruct((_NW * _BINS,), jnp.float32),
    compiler_params=_sc_params,
    scratch_types=[
        pltpu.VMEM((_BINS,), jnp.float32),
        pltpu.VMEM((2000,), jnp.int32),
    ],
)
def _deg_kernel(ei_hbm, out_hbm, bins, estage):
    c = lax.axis_index("c")
    s = lax.axis_index("s")
    wid = c * _NS + s

    zeros16 = jnp.zeros((16,), jnp.float32)
    ones16 = jnp.ones((16,), jnp.float32)

    def zb(i, carry):
        bins[pl.ds(i * 16, 16)] = zeros16
        return carry

    lax.fori_loop(0, _BINS // 16, zb, 0)

    for l in range(_L):
        for st in range(5):
            pltpu.sync_copy(
                ei_hbm.at[pl.ds(l * _E + wid * _EPW + st * 2000, 2000)],
                estage,
            )

            def hb(i, carry):
                idx = estage[pl.ds(i * 16, 16)] + l * _LPAD
                plsc.addupdate_scatter(bins, [idx], ones16)
                return carry

            lax.fori_loop(0, 125, hb, 0)

    pltpu.sync_copy(bins, out_hbm.at[pl.ds(wid * _BINS, _BINS)])


_NCH = _E // _K
_CPT = 80
_CLAST = _NCH - _CPT * (_NW - 1)


@functools.partial(
    pl.kernel,
    mesh=_mesh,
    out_type=jax.ShapeDtypeStruct((_NC * _N, _D), jnp.float32),
    compiler_params=_sc_params,
    scratch_types=[
        pltpu.VMEM((_CPT // 2, _K), jnp.int32),
        pltpu.VMEM((_CPT // 2, _K), jnp.int32),
        pltpu.VMEM((_K, _D), jnp.float32),
        pltpu.VMEM((_K, _D), jnp.float32),
        pltpu.VMEM((16, _D), jnp.float32),
        pltpu.VMEM_SHARED((_APAD, _D), jnp.float32),
        pltpu.SemaphoreType.DMA,
        pltpu.SemaphoreType.DMA,
    ],
)
def _scatter_kernel(g_hbm, src_hbm, dst_hbm, out_hbm,
                    colbuf, rowbuf, rows0, rows1, zbuf, acc, sem0, sem1):
    c = lax.axis_index("c")
    s = lax.axis_index("s")
    wid = c * _NS + s

    zeros16 = jnp.zeros((16,), jnp.float32)
    for r in range(16):
        for j in range(_D // 16):
            zbuf[r, pl.ds(j * 16, 16)] = zeros16

    stripe = _APAD // _NS

    def zs(i, carry):
        pltpu.sync_copy(zbuf, acc.at[pl.ds(s * stripe + i * 16, 16)])
        return carry

    lax.fori_loop(0, stripe // 16, zs, 0)

    is_last = wid == _NW - 1
    row0 = wid * _CPT
    half = _CPT // 2

    plsc.subcore_barrier()

    for h in range(2):
        hbase = row0 + h * half

        @pl.when(jnp.logical_not(is_last))
        def _stage_full():
            pltpu.sync_copy(src_hbm.at[pl.ds(hbase, half)], colbuf)
            pltpu.sync_copy(dst_hbm.at[pl.ds(hbase, half)], rowbuf)

        if h == 0:
            @pl.when(is_last)
            def _stage_last():
                pltpu.sync_copy(src_hbm.at[pl.ds((_NW - 1) * _CPT, _CLAST)],
                                colbuf.at[pl.ds(0, _CLAST)])
                pltpu.sync_copy(dst_hbm.at[pl.ds((_NW - 1) * _CPT, _CLAST)],
                                rowbuf.at[pl.ds(0, _CLAST)])
            npairs = jnp.where(is_last, _CLAST // 2, half // 2)
        else:
            npairs = jnp.where(is_last, 0, half // 2)

        @pl.when(npairs > 0)
        def _run():
            def body(gi, carry):
                i0 = 2 * gi
                i1 = i0 + 1
                d0 = pltpu.async_copy(g_hbm.at[colbuf.at[i0]], rows0, sem0)
                d1 = pltpu.async_copy(g_hbm.at[colbuf.at[i1]], rows1, sem1)
                d0.wait()
                pltpu.sync_copy(rows0, acc.at[rowbuf.at[i0]], add=True)
                d1.wait()
                pltpu.sync_copy(rows1, acc.at[rowbuf.at[i1]], add=True)
                return carry

            lax.fori_loop(0, npairs, body, 0)

    plsc.subcore_barrier()

    pltpu.sync_copy(
        acc.at[pl.ds(s * 624, 624)],
        out_hbm.at[pl.ds(c * _N + s * 624, 624)],
    )

    @pl.when(s == _NS - 1)
    def _wb_tail():
        pltpu.sync_copy(
            acc.at[pl.ds(624 * _NS, _N - 624 * _NS)],
            out_hbm.at[pl.ds(c * _N + 624 * _NS, _N - 624 * _NS)],
        )


_BN = 1000
_GRID = _N // _BN


def _dinv_body(degp_ref, o_ref):
    deg = jnp.sum(degp_ref[...], axis=0, keepdims=True) + 1.0
    o_ref[...] = lax.rsqrt(deg)


def _dinv_tc(degp):
    return pl.pallas_call(
        _dinv_body,
        grid=(8,),
        in_specs=[pl.BlockSpec((_NW, _BINS // 8), lambda i: (0, i))],
        out_specs=pl.BlockSpec((1, _BINS // 8), lambda i: (0, i)),
        out_shape=jax.ShapeDtypeStruct((1, _BINS), jnp.float32),
    )(degp)


def _t0_body(x_ref, w_ref, dv_ref, o_ref):
    h = jnp.dot(x_ref[...], w_ref[...], preferred_element_type=jnp.float32)
    o_ref[...] = dv_ref[...] * h


def _t0(x, w, dvc):
    return pl.pallas_call(
        _t0_body,
        grid=(_GRID,),
        in_specs=[
            pl.BlockSpec((_BN, _D), lambda i: (i, 0)),
            pl.BlockSpec((_D, _D), lambda i: (0, 0)),
            pl.BlockSpec((_BN, 1), lambda i: (i, 0)),
        ],
        out_specs=pl.BlockSpec((_BN, _D), lambda i: (i, 0)),
        out_shape=jax.ShapeDtypeStruct((_N, _D), jnp.float32),
    )(x, w, dvc)


def _tmid_body(pa_ref, pb_ref, g_ref, dva_ref, b_ref, w_ref, dvb_ref, o_ref):
    t = dva_ref[...] * (pa_ref[...] + pb_ref[...] + g_ref[...]) + b_ref[...]
    x1 = jnp.maximum(t, 0.0)
    h = jnp.dot(x1, w_ref[...], preferred_element_type=jnp.float32)
    o_ref[...] = dvb_ref[...] * h


def _tmid(pa, pb, g, dvac, b, w, dvbc):
    return pl.pallas_call(
        _tmid_body,
        grid=(_GRID,),
        in_specs=[
            pl.BlockSpec((_BN, _D), lambda i: (i, 0)),
            pl.BlockSpec((_BN, _D), lambda i: (i, 0)),
            pl.BlockSpec((_BN, _D), lambda i: (i, 0)),
            pl.BlockSpec((_BN, 1), lambda i: (i, 0)),
            pl.BlockSpec((1, _D), lambda i: (0, 0)),
            pl.BlockSpec((_D, _D), lambda i: (0, 0)),
            pl.BlockSpec((_BN, 1), lambda i: (i, 0)),
        ],
        out_specs=pl.BlockSpec((_BN, _D), lambda i: (i, 0)),
        out_shape=jax.ShapeDtypeStruct((_N, _D), jnp.float32),
    )(pa, pb, g, dvac, b, w, dvbc)


def _t3_body(pa_ref, pb_ref, g_ref, dva_ref, b_ref, o_ref):
    t = dva_ref[...] * (pa_ref[...] + pb_ref[...] + g_ref[...]) + b_ref[...]
    o_ref[...] = jnp.maximum(t, 0.0)


def _t3(pa, pb, g, dvac, b):
    return pl.pallas_call(
        _t3_body,
        grid=(_GRID,),
        in_specs=[
            pl.BlockSpec((_BN, _D), lambda i: (i, 0)),
            pl.BlockSpec((_BN, _D), lambda i: (i, 0)),
            pl.BlockSpec((_BN, _D), lambda i: (i, 0)),
            pl.BlockSpec((_BN, 1), lambda i: (i, 0)),
            pl.BlockSpec((1, _D), lambda i: (0, 0)),
        ],
        out_specs=pl.BlockSpec((_BN, _D), lambda i: (i, 0)),
        out_shape=jax.ShapeDtypeStruct((_N, _D), jnp.float32),
    )(pa, pb, g, dvac, b)


def kernel(x, edge_indices, W0, b0, W1, b1, W2, b2):
    Ws = [W0, W1, W2]
    bs = [b0.reshape(1, _D), b1.reshape(1, _D), b2.reshape(1, _D)]

    ei1cat = edge_indices[:, 1, :].reshape(_L * _E)
    degp = _deg_kernel(ei1cat)
    dinv = _dinv_tc(degp.reshape(_NW, _BINS)).reshape(_L, _LPAD)
    dcols = [dinv[l, :_N].reshape(_N, 1) for l in range(_L)]

    g = _t0(x, Ws[0], dcols[0])
    out = None
    for l in range(_L):
        p = _scatter_kernel(g,
                            edge_indices[l, 0].reshape(_NCH, _K),
                            edge_indices[l, 1].reshape(_NCH, _K))
        pa, pb = p[:_N], p[_N:]
        if l + 1 < _L:
            g = _tmid(pa, pb, g, dcols[l], bs[l], Ws[l + 1], dcols[l + 1])
        else:
            out = _t3(pa, pb, g, dcols[l], bs[l])
    return out
